# baseline (speedup 1.0000x reference)
import functools

import jax
import jax.numpy as jnp
from jax import lax
from jax.experimental import pallas as pl
from jax.experimental.pallas import tpu as pltpu
from jax.experimental.pallas import tpu_sc as plsc

_NC = 2
_NS = 16
_NW = _NC * _NS
_CHUNK = 32768


def _sc_copy(n_total: int):
    per_w = n_total // _NW
    n_chunks = per_w // _CHUNK
    mesh = plsc.VectorSubcoreMesh(core_axis_name="c", subcore_axis_name="s")

    @functools.partial(
        pl.kernel,
        mesh=mesh,
        out_type=jax.ShapeDtypeStruct((n_total,), jnp.float32),
        scratch_types=[
            pltpu.VMEM_SHARED((_NS, 2, _CHUNK), jnp.float32),
            pltpu.SemaphoreType.DMA((n_chunks,)),
            pltpu.SemaphoreType.DMA((n_chunks,)),
        ],
    )
    def k(w_hbm, out_hbm, buf, in_sems, out_sems):
        sid = lax.axis_index("s")
        wid = sid * _NC + lax.axis_index("c")
        base = wid * per_w

        def start_in(j):
            off = base + j * _CHUNK
            return pltpu.async_copy(
                w_hbm.at[pl.ds(off, _CHUNK)], buf.at[sid, j % 2], in_sems.at[j]
            )

        def start_out(j):
            off = base + j * _CHUNK
            return pltpu.async_copy(
                buf.at[sid, j % 2], out_hbm.at[pl.ds(off, _CHUNK)], out_sems.at[j]
            )

        ins = [None] * n_chunks
        outs = [None] * n_chunks
        ins[0] = start_in(0)
        for j in range(n_chunks):
            if j + 1 < n_chunks:
                if j >= 1:
                    outs[j - 1].wait()
                ins[j + 1] = start_in(j + 1)
            ins[j].wait()
            outs[j] = start_out(j)
        for j in range(max(0, n_chunks - 2), n_chunks):
            outs[j].wait()

    return k


def kernel(x, weight):
    seq_len = x.shape[1]
    d_model = weight.shape[2]
    n_total = seq_len * d_model
    flat = weight.reshape(-1)
    out = _sc_copy(n_total)(flat)
    return out.reshape(1, seq_len, d_model)


# speedup vs baseline: 1.0243x; 1.0243x over previous
import functools

import jax
import jax.numpy as jnp
from jax import lax
from jax.experimental import pallas as pl
from jax.experimental.pallas import tpu as pltpu
from jax.experimental.pallas import tpu_sc as plsc

_NC = 2
_NS = 16
_NW = _NC * _NS
_CHUNK = 16384
_NBUF = 4


def _sc_copy(n_total: int):
    per_w = n_total // _NW
    n_chunks = per_w // _CHUNK
    mesh = plsc.VectorSubcoreMesh(core_axis_name="c", subcore_axis_name="s")

    @functools.partial(
        pl.kernel,
        mesh=mesh,
        out_type=jax.ShapeDtypeStruct((n_total,), jnp.float32),
        scratch_types=[
            pltpu.VMEM((_NBUF, _CHUNK), jnp.float32),
            pltpu.SemaphoreType.DMA((n_chunks,)),
            pltpu.SemaphoreType.DMA((n_chunks,)),
        ],
    )
    def k(w_hbm, out_hbm, buf, in_sems, out_sems):
        wid = lax.axis_index("s") * _NC + lax.axis_index("c")
        base = wid * per_w

        def start_in(j):
            off = base + j * _CHUNK
            return pltpu.async_copy(
                w_hbm.at[pl.ds(off, _CHUNK)], buf.at[j % _NBUF], in_sems.at[j]
            )

        def start_out(j):
            off = base + j * _CHUNK
            return pltpu.async_copy(
                buf.at[j % _NBUF], out_hbm.at[pl.ds(off, _CHUNK)], out_sems.at[j]
            )

        ins = [None] * n_chunks
        outs = [None] * n_chunks
        for j in range(min(_NBUF, n_chunks)):
            ins[j] = start_in(j)
        for j in range(n_chunks):
            ins[j].wait()
            outs[j] = start_out(j)
            nxt = j + _NBUF
            if nxt < n_chunks:
                outs[nxt - _NBUF].wait()
                ins[nxt] = start_in(nxt)
        for j in range(max(0, n_chunks - _NBUF), n_chunks):
            if outs[j] is not None:
                outs[j].wait()

    return k


def kernel(x, weight):
    seq_len = x.shape[1]
    d_model = weight.shape[2]
    n_total = seq_len * d_model
    flat = weight.reshape(-1)
    out = _sc_copy(n_total)(flat)
    return out.reshape(1, seq_len, d_model)


# speedup vs baseline: 7.2217x; 7.0506x over previous
import jax
import jax.numpy as jnp
from jax.experimental import pallas as pl
from jax.experimental.pallas import tpu as pltpu

_N_CHUNKS = 4


def _dma_pipeline(w_ref, o_ref, buf, in_sems, out_sems):
    seq_len = o_ref.shape[1]
    chunk = seq_len // _N_CHUNKS
    ins = [
        pltpu.make_async_copy(
            w_ref.at[0, pl.ds(i * chunk, chunk), :],
            buf.at[i],
            in_sems.at[i],
        )
        for i in range(_N_CHUNKS)
    ]
    outs = [
        pltpu.make_async_copy(
            buf.at[i],
            o_ref.at[0, pl.ds(i * chunk, chunk), :],
            out_sems.at[i],
        )
        for i in range(_N_CHUNKS)
    ]
    for c in ins:
        c.start()
    for i in range(_N_CHUNKS):
        ins[i].wait()
        outs[i].start()
    for c in outs:
        c.wait()


def kernel(x, weight):
    seq_len = x.shape[1]
    d_model = weight.shape[2]
    chunk = seq_len // _N_CHUNKS
    return pl.pallas_call(
        _dma_pipeline,
        in_specs=[pl.BlockSpec(memory_space=pl.ANY)],
        out_specs=pl.BlockSpec(memory_space=pl.ANY),
        out_shape=jax.ShapeDtypeStruct((1, seq_len, d_model), weight.dtype),
        scratch_shapes=[
            pltpu.VMEM((_N_CHUNKS, chunk, d_model), weight.dtype),
            pltpu.SemaphoreType.DMA((_N_CHUNKS,)),
            pltpu.SemaphoreType.DMA((_N_CHUNKS,)),
        ],
    )(weight)


# speedup vs baseline: 7.3075x; 1.0119x over previous
import jax
import jax.numpy as jnp
from jax.experimental import pallas as pl
from jax.experimental.pallas import tpu as pltpu

_N_CHUNKS = 2


def _dma_pipeline(w_ref, o_ref, buf, in_sems, out_sems):
    seq_len = o_ref.shape[1]
    chunk = seq_len // _N_CHUNKS
    ins = [
        pltpu.make_async_copy(
            w_ref.at[0, pl.ds(i * chunk, chunk), :],
            buf.at[i],
            in_sems.at[i],
        )
        for i in range(_N_CHUNKS)
    ]
    outs = [
        pltpu.make_async_copy(
            buf.at[i],
            o_ref.at[0, pl.ds(i * chunk, chunk), :],
            out_sems.at[i],
        )
        for i in range(_N_CHUNKS)
    ]
    for c in ins:
        c.start()
    for i in range(_N_CHUNKS):
        ins[i].wait()
        outs[i].start()
    for c in outs:
        c.wait()


def kernel(x, weight):
    seq_len = x.shape[1]
    d_model = weight.shape[2]
    chunk = seq_len // _N_CHUNKS
    return pl.pallas_call(
        _dma_pipeline,
        in_specs=[pl.BlockSpec(memory_space=pl.ANY)],
        out_specs=pl.BlockSpec(memory_space=pl.ANY),
        out_shape=jax.ShapeDtypeStruct((1, seq_len, d_model), weight.dtype),
        scratch_shapes=[
            pltpu.VMEM((_N_CHUNKS, chunk, d_model), weight.dtype),
            pltpu.SemaphoreType.DMA((_N_CHUNKS,)),
            pltpu.SemaphoreType.DMA((_N_CHUNKS,)),
        ],
    )(weight)
